# baseline (speedup 1.0000x reference)
import functools

import jax
import jax.numpy as jnp
from jax import lax
from jax.experimental import pallas as pl
from jax.experimental.pallas import tpu as pltpu
from jax.experimental.pallas import tpu_sc as plsc

HIDDEN = 2048
NUM_EXPERTS = 8
TOP_K = 2
BLOCK_ROWS = 2048
TC_LANES = 128

NC = 2
NS = 16
NW = NC * NS
LANES = 16


def _logits_block(x_ref, w_ref, out_ref):
    logits_t = jax.lax.dot_general(
        w_ref[...], x_ref[...],
        dimension_numbers=(((1,), (1,)), ((), ())),
        preferred_element_type=jnp.float32,
    )
    r = x_ref.shape[0]
    parts = [logits_t[:, tb * TC_LANES:(tb + 1) * TC_LANES]
             for tb in range(r // TC_LANES)]
    out_ref[...] = jnp.concatenate(parts, axis=0)


def _tc_logits(x, gate_w, n, h):
    grid = (n // BLOCK_ROWS,)
    rows_per_block = BLOCK_ROWS * NUM_EXPERTS // TC_LANES
    return pl.pallas_call(
        _logits_block,
        grid=grid,
        in_specs=[
            pl.BlockSpec((BLOCK_ROWS, h), lambda i: (i, 0)),
            pl.BlockSpec((NUM_EXPERTS, h), lambda i: (0, 0)),
        ],
        out_specs=pl.BlockSpec((rows_per_block, TC_LANES), lambda i: (i, 0)),
        out_shape=jax.ShapeDtypeStruct(
            (n * NUM_EXPERTS // TC_LANES, TC_LANES), jnp.float32),
    )(x, gate_w)


def _make_sc_router(n):
    chunk = n // NW
    tblocks = chunk // TC_LANES
    vgroups = TC_LANES // LANES
    mesh = plsc.VectorSubcoreMesh(
        core_axis_name="c", subcore_axis_name="s",
        num_cores=NC, num_subcores=NS)

    @functools.partial(
        pl.kernel,
        mesh=mesh,
        compiler_params=pltpu.CompilerParams(needs_layout_passes=False),
        out_type=[
            jax.ShapeDtypeStruct((n * NUM_EXPERTS,), jnp.float32),
            jax.ShapeDtypeStruct((n * TOP_K,), jnp.int32),
            jax.ShapeDtypeStruct((n * TOP_K,), jnp.float32),
        ],
        scratch_types=[
            pltpu.VMEM((chunk * NUM_EXPERTS,), jnp.float32),
            pltpu.VMEM((chunk * NUM_EXPERTS,), jnp.float32),
            pltpu.VMEM((chunk * TOP_K,), jnp.int32),
            pltpu.VMEM((chunk * TOP_K,), jnp.float32),
        ],
    )
    def sc_router(lg_hbm, probs_hbm, idx_hbm, wts_hbm, lg_v, p_v, i_v, w_v):
        wid = lax.axis_index("s") * NC + lax.axis_index("c")
        base = wid * chunk
        pltpu.sync_copy(
            lg_hbm.at[pl.ds(base * NUM_EXPERTS, chunk * NUM_EXPERTS)], lg_v)

        lane = lax.iota(jnp.int32, LANES)
        neg1 = jnp.full((LANES,), -1.0, jnp.float32)

        @pl.loop(0, 1)
        def _(tb):
            tb_off = tb * (TC_LANES * NUM_EXPERTS)

            @pl.loop(0, vgroups)
            def _(v):
                off = tb_off + v * LANES
                ls = [lg_v[pl.ds(off + e * TC_LANES, LANES)]
                      for e in range(NUM_EXPERTS)]
                m = ls[0]
                for e in range(1, NUM_EXPERTS):
                    m = jnp.maximum(m, ls[e])
                es = [jnp.exp(l - m) for l in ls]
                ssum = es[0]
                for e in range(1, NUM_EXPERTS):
                    ssum = ssum + es[e]
                ps = [x / ssum for x in es]

                v1 = ps[0]
                i1 = jnp.zeros((LANES,), jnp.int32)
                for e in range(1, NUM_EXPERTS):
                    e_vec = jnp.full((LANES,), e, jnp.int32)
                    gt = ps[e] > v1
                    v1 = jnp.where(gt, ps[e], v1)
                    i1 = jnp.where(gt, e_vec, i1)
                v2 = jnp.where(i1 == 0, neg1, ps[0])
                i2 = jnp.zeros((LANES,), jnp.int32)
                for e in range(1, NUM_EXPERTS):
                    e_vec = jnp.full((LANES,), e, jnp.int32)
                    pe = jnp.where(i1 == e_vec, neg1, ps[e])
                    gt = pe > v2
                    v2 = jnp.where(gt, pe, v2)
                    i2 = jnp.where(gt, e_vec, i2)

                tok = tb * TC_LANES + v * LANES + lane
                tok8 = tok * NUM_EXPERTS
                for e in range(NUM_EXPERTS):
                    plsc.store_scatter(p_v, [tok8 + e], ps[e])
                tok2 = tok * TOP_K
                plsc.store_scatter(i_v, [tok2], i1)
                plsc.store_scatter(i_v, [tok2 + 1], i2)
                denom = v1 + v2
                plsc.store_scatter(w_v, [tok2], v1 / denom)
                plsc.store_scatter(w_v, [tok2 + 1], v2 / denom)

        pltpu.sync_copy(p_v, probs_hbm.at[pl.ds(base * NUM_EXPERTS,
                                                chunk * NUM_EXPERTS)])
        pltpu.sync_copy(i_v, idx_hbm.at[pl.ds(base * TOP_K, chunk * TOP_K)])
        pltpu.sync_copy(w_v, wts_hbm.at[pl.ds(base * TOP_K, chunk * TOP_K)])

    return sc_router


@jax.jit
def kernel(hidden_states, gate_w):
    b, s, h = hidden_states.shape
    n = b * s
    x = hidden_states.reshape(n, h)

    logits_f = _tc_logits(x, gate_w, n, h)
    probs_f, idx_f, wts_f = _make_sc_router(n)(logits_f.reshape(-1))

    return (
        probs_f.reshape(b, s, NUM_EXPERTS),
        idx_f.reshape(b, s, TOP_K),
        wts_f.reshape(b, s, TOP_K),
    )


# speedup vs baseline: 2.1660x; 2.1660x over previous
import functools

import jax
import jax.numpy as jnp
from jax.experimental import pallas as pl

HIDDEN = 2048
NUM_EXPERTS = 8
TOP_K = 2
BLOCK_ROWS = 2048


def _router_block(x_ref, w_ref, probs_ref, idx_ref, wts_ref):
    logits_t = jax.lax.dot_general(
        w_ref[...], x_ref[...],
        dimension_numbers=(((1,), (1,)), ((), ())),
        preferred_element_type=jnp.float32,
    )
    m = jnp.max(logits_t, axis=0, keepdims=True)
    e = jnp.exp(logits_t - m)
    s = jnp.sum(e, axis=0, keepdims=True)
    probs_t = e / s

    iota = jax.lax.broadcasted_iota(jnp.int32, probs_t.shape, 0)
    v1 = jnp.max(probs_t, axis=0, keepdims=True)
    i1 = jnp.min(jnp.where(probs_t == v1, iota, NUM_EXPERTS), axis=0,
                 keepdims=True)
    masked = jnp.where(iota == i1, -jnp.inf, probs_t)
    v2 = jnp.max(masked, axis=0, keepdims=True)
    i2 = jnp.min(jnp.where(masked == v2, iota, NUM_EXPERTS), axis=0,
                 keepdims=True)

    probs_ref[...] = probs_t
    idx_ref[...] = jnp.concatenate([i1, i2], axis=0)
    denom = v1 + v2
    wts_ref[...] = jnp.concatenate([v1 / denom, v2 / denom], axis=0)


@functools.partial(jax.jit, static_argnames=("interpret",))
def kernel(hidden_states, gate_w, interpret=False):
    b, s, h = hidden_states.shape
    n = b * s
    x = hidden_states.reshape(n, h)

    grid = (n // BLOCK_ROWS,)
    probs_t, idx_t, wts_t = pl.pallas_call(
        _router_block,
        grid=grid,
        in_specs=[
            pl.BlockSpec((BLOCK_ROWS, h), lambda i: (i, 0)),
            pl.BlockSpec((NUM_EXPERTS, h), lambda i: (0, 0)),
        ],
        out_specs=[
            pl.BlockSpec((NUM_EXPERTS, BLOCK_ROWS), lambda i: (0, i)),
            pl.BlockSpec((TOP_K, BLOCK_ROWS), lambda i: (0, i)),
            pl.BlockSpec((TOP_K, BLOCK_ROWS), lambda i: (0, i)),
        ],
        out_shape=[
            jax.ShapeDtypeStruct((NUM_EXPERTS, n), jnp.float32),
            jax.ShapeDtypeStruct((TOP_K, n), jnp.int32),
            jax.ShapeDtypeStruct((TOP_K, n), jnp.float32),
        ],
        interpret=interpret,
    )(x, gate_w)

    return (
        probs_t.T.reshape(b, s, NUM_EXPERTS),
        idx_t.T.reshape(b, s, TOP_K),
        wts_t.T.reshape(b, s, TOP_K),
    )


# speedup vs baseline: 2.1727x; 1.0031x over previous
import functools

import jax
import jax.numpy as jnp
from jax.experimental import pallas as pl

HIDDEN = 2048
NUM_EXPERTS = 8
TOP_K = 2
BLOCK_ROWS = 1024


def _router_block(x_ref, w_ref, probs_ref, idx_ref, wts_ref):
    logits_t = jax.lax.dot_general(
        w_ref[...], x_ref[...],
        dimension_numbers=(((1,), (1,)), ((), ())),
        preferred_element_type=jnp.float32,
    )
    m = jnp.max(logits_t, axis=0, keepdims=True)
    e = jnp.exp(logits_t - m)
    s = jnp.sum(e, axis=0, keepdims=True)
    probs_t = e / s

    iota = jax.lax.broadcasted_iota(jnp.int32, probs_t.shape, 0)
    v1 = jnp.max(probs_t, axis=0, keepdims=True)
    i1 = jnp.min(jnp.where(probs_t == v1, iota, NUM_EXPERTS), axis=0,
                 keepdims=True)
    masked = jnp.where(iota == i1, -jnp.inf, probs_t)
    v2 = jnp.max(masked, axis=0, keepdims=True)
    i2 = jnp.min(jnp.where(masked == v2, iota, NUM_EXPERTS), axis=0,
                 keepdims=True)

    probs_ref[...] = probs_t
    idx_ref[...] = jnp.concatenate([i1, i2], axis=0)
    denom = v1 + v2
    wts_ref[...] = jnp.concatenate([v1 / denom, v2 / denom], axis=0)


@functools.partial(jax.jit, static_argnames=("interpret",))
def kernel(hidden_states, gate_w, interpret=False):
    b, s, h = hidden_states.shape
    n = b * s
    x = hidden_states.reshape(n, h)

    grid = (n // BLOCK_ROWS,)
    probs_t, idx_t, wts_t = pl.pallas_call(
        _router_block,
        grid=grid,
        in_specs=[
            pl.BlockSpec((BLOCK_ROWS, h), lambda i: (i, 0)),
            pl.BlockSpec((NUM_EXPERTS, h), lambda i: (0, 0)),
        ],
        out_specs=[
            pl.BlockSpec((NUM_EXPERTS, BLOCK_ROWS), lambda i: (0, i)),
            pl.BlockSpec((TOP_K, BLOCK_ROWS), lambda i: (0, i)),
            pl.BlockSpec((TOP_K, BLOCK_ROWS), lambda i: (0, i)),
        ],
        out_shape=[
            jax.ShapeDtypeStruct((NUM_EXPERTS, n), jnp.float32),
            jax.ShapeDtypeStruct((TOP_K, n), jnp.int32),
            jax.ShapeDtypeStruct((TOP_K, n), jnp.float32),
        ],
        interpret=interpret,
    )(x, gate_w)

    return (
        probs_t.T.reshape(b, s, NUM_EXPERTS),
        idx_t.T.reshape(b, s, TOP_K),
        wts_t.T.reshape(b, s, TOP_K),
    )
